# baseline (speedup 1.0000x reference)
import functools

import jax
import jax.numpy as jnp
from jax import lax
from jax.experimental import pallas as pl
from jax.experimental.pallas import tpu as pltpu
from jax.experimental.pallas import tpu_sc as plsc

CHUNK = 128
L = 16


def kernel(dom_idx, paras):
  B = dom_idx.shape[0]
  V, D = paras.shape
  table = paras.reshape(V // 2, 2 * D)
  info = plsc.get_sparse_core_info()
  nc = info.num_cores
  nw = nc * info.num_subcores
  b_per_w = B // nw
  n_chunks = b_per_w // CHUNK
  n_groups = b_per_w // L

  mesh = plsc.VectorSubcoreMesh(core_axis_name="c", subcore_axis_name="s")

  @functools.partial(
      pl.kernel,
      mesh=mesh,
      out_type=jax.ShapeDtypeStruct((D, B), jnp.float32),
      compiler_params=pltpu.CompilerParams(needs_layout_passes=False),
      scratch_types=[
          pltpu.VMEM((b_per_w,), jnp.int32),
          pltpu.VMEM((b_per_w,), jnp.int32),
          pltpu.VMEM((b_per_w, 2 * D), jnp.float32),
          pltpu.VMEM((D, b_per_w), jnp.float32),
          pltpu.SemaphoreType.DMA,
      ],
  )
  def gather_kernel(idx_hbm, table_hbm, out_hbm, idx_v, idx2_v, rows_v,
                    out_v, sem):
    wid = lax.axis_index("s") * nc + lax.axis_index("c")
    base = wid * b_per_w
    pltpu.sync_copy(idx_hbm.at[pl.ds(base, b_per_w)], idx_v)

    def halve(g, carry):
      v = idx_v[pl.ds(g * L, L)]
      idx2_v[pl.ds(g * L, L)] = lax.shift_right_logical(v, 1)
      return carry

    lax.fori_loop(0, n_groups, halve, 0)

    for j in range(n_chunks):
      pltpu.async_copy(
          table_hbm.at[idx2_v.at[pl.ds(j * CHUNK, CHUNK)]],
          rows_v.at[pl.ds(j * CHUNK, CHUNK)],
          sem,
      )
    pltpu.make_async_copy(
        table_hbm.at[pl.ds(0, b_per_w)], rows_v, sem
    ).wait()

    iota = lax.iota(jnp.int32, L)

    def compact(g, carry):
      rowv = iota + g * L
      par = jnp.bitwise_and(idx_v[pl.ds(g * L, L)], 1)
      colbase = lax.shift_left(par, 6)
      for c in range(D):
        val = plsc.load_gather(rows_v, [rowv, colbase + c])
        out_v[c, pl.ds(g * L, L)] = val
      return carry

    lax.fori_loop(0, n_groups, compact, 0)
    pltpu.sync_copy(out_v, out_hbm.at[:, pl.ds(base, b_per_w)])

  out_t = gather_kernel(dom_idx, table)
  return out_t.T


# speedup vs baseline: 1.6624x; 1.6624x over previous
import functools

import jax
import jax.numpy as jnp
from jax import lax
from jax.experimental import pallas as pl
from jax.experimental.pallas import tpu as pltpu
from jax.experimental.pallas import tpu_sc as plsc

L = 16


def kernel(dom_idx, paras):
  B = dom_idx.shape[0]
  V, D = paras.shape
  info = plsc.get_sparse_core_info()
  nc = info.num_cores
  nw = nc * info.num_subcores
  b_per_w = B // nw
  n_groups = b_per_w // L

  mesh = plsc.VectorSubcoreMesh(core_axis_name="c", subcore_axis_name="s")

  @functools.partial(
      pl.kernel,
      mesh=mesh,
      out_type=jax.ShapeDtypeStruct((B, D), jnp.float32),
      compiler_params=pltpu.CompilerParams(needs_layout_passes=False),
      scratch_types=[
          pltpu.VMEM((b_per_w,), jnp.int32),
          pltpu.VMEM((b_per_w, D), jnp.float32),
          pltpu.SemaphoreType.DMA,
      ],
  )
  def gather_kernel(idx_hbm, table_hbm, out_hbm, idx_v, rows_v, sem):
    wid = lax.axis_index("s") * nc + lax.axis_index("c")
    base = wid * b_per_w
    pltpu.sync_copy(idx_hbm.at[pl.ds(base, b_per_w)], idx_v)
    iota = lax.iota(jnp.int32, L)

    def issue_group(g, carry):
      v = idx_v[pl.ds(g * L, L)]
      for l in range(L):
        s = lax.reduce_max(jnp.where(iota == l, v, 0), axes=(0,))
        pltpu.async_copy(
            table_hbm.at[pl.ds(s, 1)],
            rows_v.at[pl.ds(g * L + l, 1)],
            sem,
        )
      return carry

    lax.fori_loop(0, n_groups, issue_group, 0)
    pltpu.make_async_copy(
        table_hbm.at[pl.ds(0, b_per_w)], rows_v, sem
    ).wait()
    pltpu.sync_copy(rows_v, out_hbm.at[pl.ds(base, b_per_w)])

  return gather_kernel(dom_idx, paras)
